# Initial kernel scaffold; baseline (speedup 1.0000x reference)
import functools

import jax
import jax.numpy as jnp
from jax import lax
from jax.experimental import pallas as pl
from jax.experimental.pallas import tpu as pltpu
from jax.experimental.pallas import tpu_sc as plsc

_N = 10000
_D = 128
_E = 320000
_NC = 2
_NS = 16
_NW = _NC * _NS
_EPW = _E // _NW
_WIN = 80
_NWIN = _EPW // _WIN
_RPS = _N // _NS
_ZROWS = 125
_LANES = 16

_BM = 400


def _mm_body(x_ref, w_ref, o_ref):
    o_ref[...] = lax.dot_general(
        x_ref[...], w_ref[...], (((1,), (1,)), ((), ())),
        precision=lax.Precision.HIGHEST, preferred_element_type=jnp.float32)


def _tc_matmul(x, w):
    return pl.pallas_call(
        _mm_body,
        grid=(_N // _BM,),
        in_specs=[pl.BlockSpec((_BM, _D), lambda i: (i, 0)),
                  pl.BlockSpec((_D, _D), lambda i: (0, 0))],
        out_specs=pl.BlockSpec((_BM, _D), lambda i: (i, 0)),
        out_shape=jax.ShapeDtypeStruct((_N, _D), jnp.float32),
    )(x, w)


def _fuse_body(p_ref, w_ref, o_ref):
    h = jnp.maximum(p_ref[0] + p_ref[1], 0.0)
    o_ref[...] = lax.dot_general(
        h, w_ref[...], (((1,), (1,)), ((), ())),
        precision=lax.Precision.HIGHEST, preferred_element_type=jnp.float32)


def _tc_relu_add_matmul(p, w):
    return pl.pallas_call(
        _fuse_body,
        grid=(_N // _BM,),
        in_specs=[pl.BlockSpec((_NC, _BM, _D), lambda i: (0, i, 0)),
                  pl.BlockSpec((_D, _D), lambda i: (0, 0))],
        out_specs=pl.BlockSpec((_BM, _D), lambda i: (i, 0)),
        out_shape=jax.ShapeDtypeStruct((_N, _D), jnp.float32),
    )(p, w)


def _add_body(q_ref, o_ref):
    o_ref[...] = q_ref[0] + q_ref[1]


def _tc_add(q):
    return pl.pallas_call(
        _add_body,
        grid=(_N // _BM,),
        in_specs=[pl.BlockSpec((_NC, _BM, _D), lambda i: (0, i, 0))],
        out_specs=pl.BlockSpec((_BM, _D), lambda i: (i, 0)),
        out_shape=jax.ShapeDtypeStruct((_N, _D), jnp.float32),
    )(q)


def _sc_spmm(h, row3, col2, ev2):
    mesh = plsc.VectorSubcoreMesh(core_axis_name="c", subcore_axis_name="s")

    @functools.partial(
        pl.kernel,
        out_type=jax.ShapeDtypeStruct((_NC, _N, _D), jnp.float32),
        mesh=mesh,
        scratch_types=[
            pltpu.VMEM((_EPW,), jnp.int32),
            pltpu.VMEM((_NWIN, _WIN), jnp.int32),
            pltpu.VMEM((_EPW,), jnp.float32),
            pltpu.VMEM((_WIN, _D), jnp.float32),
            pltpu.VMEM((_ZROWS, _D), jnp.float32),
            pltpu.VMEM_SHARED((_N, _D), jnp.float32),
        ],
    )
    def spmm(h_hbm, row_hbm, col_hbm, ev_hbm, out_hbm,
             col_v, row_v, ev_v, gbuf, zbuf, acc):
        c = lax.axis_index("c")
        s = lax.axis_index("s")
        wid = s * _NC + c

        pltpu.sync_copy(col_hbm.at[wid], col_v)
        pltpu.sync_copy(row_hbm.at[wid], row_v)
        pltpu.sync_copy(ev_hbm.at[wid], ev_v)

        @pl.loop(0, _ZROWS)
        def _zero_stage(i):
            for j in range(0, _D, _LANES):
                zbuf[pl.ds(i, 1), pl.ds(j, _LANES)] = jnp.zeros(
                    (1, _LANES), jnp.float32)

        @pl.loop(0, _RPS // _ZROWS)
        def _zero_acc(i):
            pltpu.sync_copy(zbuf, acc.at[pl.ds(s * _RPS + i * _ZROWS, _ZROWS)])

        plsc.subcore_barrier()

        @pl.loop(0, _NWIN)
        def _window(w):
            pltpu.sync_copy(h_hbm.at[col_v.at[pl.ds(w * _WIN, _WIN)]], gbuf)

            @pl.loop(0, _WIN)
            def _edge(e):
                idx16 = jnp.full((_LANES,), w * _WIN + e, jnp.int32)
                vs = plsc.load_gather(ev_v, [idx16]).reshape(1, _LANES)
                for j in range(0, _D, _LANES):
                    gbuf[pl.ds(e, 1), pl.ds(j, _LANES)] = (
                        gbuf[pl.ds(e, 1), pl.ds(j, _LANES)] * vs)

            pltpu.sync_copy(gbuf, acc.at[row_v.at[w]], add=True)

        plsc.subcore_barrier()

        @pl.loop(0, _RPS // _ZROWS)
        def _flush(i):
            r0 = s * _RPS + i * _ZROWS
            pltpu.sync_copy(acc.at[pl.ds(r0, _ZROWS)],
                            out_hbm.at[c, pl.ds(r0, _ZROWS)])

    return spmm(h, row3, col2, ev2)


def kernel(x, edge_index, edge_values, W1, W2):
    row3 = edge_index[0].reshape(_NW, _NWIN, _WIN)
    col2 = edge_index[1].reshape(_NW, _EPW)
    ev2 = edge_values.reshape(_NW, _EPW)

    h1 = _tc_matmul(x, W1)
    p = _sc_spmm(h1, row3, col2, ev2)
    h2 = _tc_relu_add_matmul(p, W2)
    q = _sc_spmm(h2, row3, col2, ev2)
    return _tc_add(q)


# speedup vs baseline: 5.2615x; 5.2615x over previous
import dataclasses
import functools

import jax
import jax.numpy as jnp
from jax import lax
from jax.experimental import pallas as pl
from jax.experimental.pallas import tpu as pltpu
from jax.experimental.pallas import tpu_sc as plsc

_N = 10000
_D = 128
_E = 320000
_NC = 2
_NS = 16
_NW = _NC * _NS
_EPW = _E // _NW
_WIN = 80
_NWIN = _EPW // _WIN
_CHUNK = 200
_NCHUNK = _N // _CHUNK
_NZCHUNK = _N // _WIN
_LANES = 16

_BM = 400


def _mm_body(x_ref, w_ref, o_ref):
    o_ref[...] = lax.dot_general(
        x_ref[...], w_ref[...], (((1,), (1,)), ((), ())),
        precision=lax.Precision.HIGHEST, preferred_element_type=jnp.float32)


def _tc_matmul(x, w):
    return pl.pallas_call(
        _mm_body,
        grid=(_N // _BM,),
        in_specs=[pl.BlockSpec((_BM, _D), lambda i: (i, 0)),
                  pl.BlockSpec((_D, _D), lambda i: (0, 0))],
        out_specs=pl.BlockSpec((_BM, _D), lambda i: (i, 0)),
        out_shape=jax.ShapeDtypeStruct((_N, _D), jnp.float32),
    )(x, w)


def _fuse_body(p_ref, w_ref, o_ref):
    h = jnp.maximum(p_ref[0] + p_ref[1], 0.0)
    o_ref[...] = lax.dot_general(
        h, w_ref[...], (((1,), (1,)), ((), ())),
        precision=lax.Precision.HIGHEST, preferred_element_type=jnp.float32)


def _tc_relu_add_matmul(p, w):
    return pl.pallas_call(
        _fuse_body,
        grid=(_N // _BM,),
        in_specs=[pl.BlockSpec((_NC, _BM, _D), lambda i: (0, i, 0)),
                  pl.BlockSpec((_D, _D), lambda i: (0, 0))],
        out_specs=pl.BlockSpec((_BM, _D), lambda i: (i, 0)),
        out_shape=jax.ShapeDtypeStruct((_N, _D), jnp.float32),
    )(p, w)


def _add_body(q_ref, o_ref):
    o_ref[...] = q_ref[0] + q_ref[1]


def _tc_add(q):
    return pl.pallas_call(
        _add_body,
        grid=(_N // _BM,),
        in_specs=[pl.BlockSpec((_NC, _BM, _D), lambda i: (0, i, 0))],
        out_specs=pl.BlockSpec((_BM, _D), lambda i: (i, 0)),
        out_shape=jax.ShapeDtypeStruct((_N, _D), jnp.float32),
    )(q)


def _sc_spmm(h, row3, col2, ev2):
    mesh = plsc.VectorSubcoreMesh(core_axis_name="c", subcore_axis_name="s")
    cp = pltpu.CompilerParams()
    if "needs_layout_passes" in pltpu.CompilerParams.__dataclass_fields__:
        cp = dataclasses.replace(cp, needs_layout_passes=False)

    @functools.partial(
        pl.kernel,
        out_type=jax.ShapeDtypeStruct((_NC, _N, _D), jnp.float32),
        mesh=mesh,
        compiler_params=cp,
        scratch_types=[
            pltpu.VMEM((_EPW,), jnp.int32),
            pltpu.VMEM((_NWIN, _WIN), jnp.int32),
            pltpu.VMEM((_EPW,), jnp.float32),
            pltpu.VMEM((_WIN, _D), jnp.float32),
            pltpu.VMEM_SHARED((_N, _D), jnp.float32),
        ],
    )
    def spmm(h_hbm, row_hbm, col_hbm, ev_hbm, out_hbm,
             col_v, row_v, ev_v, gbuf, acc):
        c = lax.axis_index("c")
        s = lax.axis_index("s")
        wid = s * _NC + c

        pltpu.sync_copy(col_hbm.at[wid], col_v)
        pltpu.sync_copy(row_hbm.at[wid], row_v)
        pltpu.sync_copy(ev_hbm.at[wid], ev_v)

        @pl.loop(0, _WIN)
        def _zero_stage(i):
            for j in range(0, _D, _LANES):
                gbuf[i, pl.ds(j, _LANES)] = jnp.zeros((_LANES,), jnp.float32)

        @pl.loop(0, pl.cdiv(_NZCHUNK, _NS))
        def _zero_acc(k):
            chunk = s + k * _NS

            @pl.when(chunk < _NZCHUNK)
            def _():
                pltpu.sync_copy(gbuf, acc.at[pl.ds(chunk * _WIN, _WIN)])

        plsc.subcore_barrier()

        @pl.loop(0, _NWIN)
        def _window(w):
            pltpu.sync_copy(h_hbm.at[col_v.at[pl.ds(w * _WIN, _WIN)]], gbuf)

            @pl.loop(0, _WIN)
            def _edge(e):
                idx16 = jnp.full((_LANES,), w * _WIN + e, jnp.int32)
                vs = plsc.load_gather(ev_v, [idx16])
                for j in range(0, _D, _LANES):
                    gbuf[e, pl.ds(j, _LANES)] = gbuf[e, pl.ds(j, _LANES)] * vs

            pltpu.sync_copy(gbuf, acc.at[row_v.at[w]], add=True)

        plsc.subcore_barrier()

        @pl.loop(0, pl.cdiv(_NCHUNK, _NS))
        def _flush(k):
            chunk = s + k * _NS

            @pl.when(chunk < _NCHUNK)
            def _():
                r0 = chunk * _CHUNK
                pltpu.sync_copy(acc.at[pl.ds(r0, _CHUNK)],
                                out_hbm.at[c, pl.ds(r0, _CHUNK)])

    return spmm(h, row3, col2, ev2)


def kernel(x, edge_index, edge_values, W1, W2):
    row3 = edge_index[0].reshape(_NW, _NWIN, _WIN)
    col2 = edge_index[1].reshape(_NW, _EPW)
    ev2 = edge_values.reshape(_NW, _EPW)

    h1 = _tc_matmul(x, W1)
    p = _sc_spmm(h1, row3, col2, ev2)
    h2 = _tc_relu_add_matmul(p, W2)
    q = _sc_spmm(h2, row3, col2, ev2)
    return _tc_add(q)


# speedup vs baseline: 6.4798x; 1.2316x over previous
import dataclasses
import functools

import jax
import jax.numpy as jnp
from jax import lax
from jax.experimental import pallas as pl
from jax.experimental.pallas import tpu as pltpu
from jax.experimental.pallas import tpu_sc as plsc

_N = 10000
_D = 128
_E = 320000
_NC = 2
_NS = 16
_NW = _NC * _NS
_EPW = _E // _NW
_WIN = 80
_EPP = 10080
_PAD = _EPP - _EPW
_NPH = 3
_EPH = _EPP // _NPH
_WPH = _EPH // _WIN
_NBUF = 3
_CHUNK = 200
_NCHUNK = _N // _CHUNK
_NZCHUNK = _N // _WIN
_LANES = 16

_BM = 400


def _mm_body(x_ref, w_ref, o_ref):
    o_ref[...] = lax.dot_general(
        x_ref[...], w_ref[...], (((1,), (1,)), ((), ())),
        precision=lax.Precision.HIGHEST, preferred_element_type=jnp.float32)


def _tc_matmul(x, w):
    return pl.pallas_call(
        _mm_body,
        grid=(_N // _BM,),
        in_specs=[pl.BlockSpec((_BM, _D), lambda i: (i, 0)),
                  pl.BlockSpec((_D, _D), lambda i: (0, 0))],
        out_specs=pl.BlockSpec((_BM, _D), lambda i: (i, 0)),
        out_shape=jax.ShapeDtypeStruct((_N, _D), jnp.float32),
    )(x, w)


def _fuse_body(p_ref, w_ref, o_ref):
    h = jnp.maximum(p_ref[0] + p_ref[1], 0.0)
    o_ref[...] = lax.dot_general(
        h, w_ref[...], (((1,), (1,)), ((), ())),
        precision=lax.Precision.HIGHEST, preferred_element_type=jnp.float32)


def _tc_relu_add_matmul(p, w):
    return pl.pallas_call(
        _fuse_body,
        grid=(_N // _BM,),
        in_specs=[pl.BlockSpec((_NC, _BM, _D), lambda i: (0, i, 0)),
                  pl.BlockSpec((_D, _D), lambda i: (0, 0))],
        out_specs=pl.BlockSpec((_BM, _D), lambda i: (i, 0)),
        out_shape=jax.ShapeDtypeStruct((_N, _D), jnp.float32),
    )(p, w)


def _add_body(q_ref, o_ref):
    o_ref[...] = q_ref[0] + q_ref[1]


def _tc_add(q):
    return pl.pallas_call(
        _add_body,
        grid=(_N // _BM,),
        in_specs=[pl.BlockSpec((_NC, _BM, _D), lambda i: (0, i, 0))],
        out_specs=pl.BlockSpec((_BM, _D), lambda i: (i, 0)),
        out_shape=jax.ShapeDtypeStruct((_N, _D), jnp.float32),
    )(q)


def _sc_spmm(h, row3, col2, ev2):
    mesh = plsc.VectorSubcoreMesh(core_axis_name="c", subcore_axis_name="s")
    cp = pltpu.CompilerParams()
    if "needs_layout_passes" in pltpu.CompilerParams.__dataclass_fields__:
        cp = dataclasses.replace(cp, needs_layout_passes=False)

    @functools.partial(
        pl.kernel,
        out_type=jax.ShapeDtypeStruct((_NC, _N, _D), jnp.float32),
        mesh=mesh,
        compiler_params=cp,
        scratch_types=[
            pltpu.VMEM((_EPH,), jnp.int32),
            pltpu.VMEM((_WPH, _WIN), jnp.int32),
            pltpu.VMEM((_EPH,), jnp.float32),
            pltpu.VMEM((_NBUF, _WIN, _D), jnp.float32),
            pltpu.VMEM_SHARED((_N, _D), jnp.float32),
            pltpu.SemaphoreType.DMA,
            pltpu.SemaphoreType.DMA,
            pltpu.SemaphoreType.DMA,
            pltpu.SemaphoreType.DMA,
            pltpu.SemaphoreType.DMA,
            pltpu.SemaphoreType.DMA,
        ],
    )
    def spmm(h_hbm, row_hbm, col_hbm, ev_hbm, out_hbm,
             col_v, row_v, ev_v, gbuf, acc,
             gs0, gs1, gs2, ss0, ss1, ss2):
        c = lax.axis_index("c")
        s = lax.axis_index("s")
        wid = s * _NC + c
        gsem = (gs0, gs1, gs2)
        ssem = (ss0, ss1, ss2)

        zb = gbuf.at[0]

        @pl.loop(0, _WIN)
        def _zero_stage(i):
            for j in range(0, _D, _LANES):
                zb[i, pl.ds(j, _LANES)] = jnp.zeros((_LANES,), jnp.float32)

        @pl.loop(0, pl.cdiv(_NZCHUNK, _NS))
        def _zero_acc(k):
            chunk = s + k * _NS

            @pl.when(chunk < _NZCHUNK)
            def _():
                pltpu.sync_copy(zb, acc.at[pl.ds(chunk * _WIN, _WIN)])

        plsc.subcore_barrier()

        def start_gather(w, b):
            pltpu.async_copy(
                h_hbm.at[col_v.at[pl.ds(w * _WIN, _WIN)]], gbuf.at[b],
                gsem[b])

        def drain(sem, b):
            pltpu.make_async_copy(
                h_hbm.at[pl.ds(0, _WIN)], gbuf.at[b], sem).wait()

        def scale(w, b):
            gb = gbuf.at[b]

            @pl.loop(0, _WIN, step=2)
            def _edge(e):
                for u in range(2):
                    idx16 = jnp.full((_LANES,), w * _WIN + e + u, jnp.int32)
                    vs = plsc.load_gather(ev_v, [idx16])
                    for j in range(0, _D, _LANES):
                        gb[e + u, pl.ds(j, _LANES)] = (
                            gb[e + u, pl.ds(j, _LANES)] * vs)

        def start_scatter(w, b):
            pltpu.async_copy(gbuf.at[b], acc.at[row_v.at[w]], ssem[b],
                             add=True)

        for ph in range(_NPH):
            pltpu.sync_copy(col_hbm.at[wid * _NPH + ph], col_v)
            pltpu.sync_copy(row_hbm.at[wid * _NPH + ph], row_v)
            pltpu.sync_copy(ev_hbm.at[wid * _NPH + ph], ev_v)
            for b in range(_NBUF):
                start_gather(b, b)

            @pl.loop(0, _WPH, step=_NBUF)
            def _window(w):
                for b in range(_NBUF):
                    drain(gsem[b], b)
                    scale(w + b, b)
                    start_scatter(w + b, b)
                    if b >= 1:
                        pb = b - 1
                        drain(ssem[pb], pb)

                        @pl.when(w + _NBUF + pb < _WPH)
                        def _():
                            start_gather(w + _NBUF + pb, pb)
                drain(ssem[_NBUF - 1], _NBUF - 1)

                @pl.when(w + 2 * _NBUF - 1 < _WPH)
                def _():
                    start_gather(w + 2 * _NBUF - 1, _NBUF - 1)

        plsc.subcore_barrier()

        @pl.loop(0, pl.cdiv(_NCHUNK, _NS))
        def _flush(k):
            chunk = s + k * _NS

            @pl.when(chunk < _NCHUNK)
            def _():
                r0 = chunk * _CHUNK
                pltpu.sync_copy(acc.at[pl.ds(r0, _CHUNK)],
                                out_hbm.at[c, pl.ds(r0, _CHUNK)])

    return spmm(h, row3, col2, ev2)


def kernel(x, edge_index, edge_values, W1, W2):
    pad2 = ((0, 0), (0, _PAD))
    row3 = jnp.pad(edge_index[0].reshape(_NW, _EPW), pad2).reshape(
        _NW * _NPH, _WPH, _WIN)
    col2 = jnp.pad(edge_index[1].reshape(_NW, _EPW), pad2).reshape(
        _NW * _NPH, _EPH)
    ev2 = jnp.pad(edge_values.reshape(_NW, _EPW), pad2).reshape(
        _NW * _NPH, _EPH)

    h1 = _tc_matmul(x, W1)
    p = _sc_spmm(h1, row3, col2, ev2)
    h2 = _tc_relu_add_matmul(p, W2)
    q = _sc_spmm(h2, row3, col2, ev2)
    return _tc_add(q)
